# baseline (speedup 1.0000x reference)
import functools

import jax
import jax.numpy as jnp
from jax import lax
from jax.experimental import pallas as pl
from jax.experimental.pallas import tpu as pltpu
from jax.experimental.pallas import tpu_sc as plsc

B, CIN, HW, P = 16, 3, 224, 14
HP = HW // P
HID, CODE_DIM, K = 96, 32, 8192
N = B * HP * HP
D = CIN * P * P
TT = 256
NT = N // TT
KC = 2048

SC_CORES, SC_SUBCORES = 2, 16
NW = SC_CORES * SC_SUBCORES
BPW = N // NW


def _enc_body(p_ref, w1_ref, b1_ref, w2_ref, b2_ref, cbt_ref, cnorm_ref,
              z_ref, idx_ref):
    p = p_ref[...]
    h = jnp.maximum(
        jnp.dot(p, w1_ref[...], preferred_element_type=jnp.float32)
        + b1_ref[...], 0.0)
    z = (jnp.dot(h, w2_ref[...], preferred_element_type=jnp.float32)
         + b2_ref[...])
    z_ref[...] = z
    znorm = jnp.sum(z * z, axis=1, keepdims=True)
    best_d = jnp.full((TT, 1), jnp.inf, jnp.float32)
    best_i = jnp.zeros((TT, 1), jnp.int32)
    for c in range(K // KC):
        cross = jnp.dot(z, cbt_ref[:, c * KC:(c + 1) * KC],
                        preferred_element_type=jnp.float32)
        d = (znorm - 2.0 * cross) + cnorm_ref[:, c * KC:(c + 1) * KC]
        m = jnp.min(d, axis=1, keepdims=True)
        ii = lax.broadcasted_iota(jnp.int32, d.shape, 1) + c * KC
        i = jnp.min(jnp.where(d == m, ii, K), axis=1, keepdims=True)
        take = m < best_d
        best_d = jnp.where(take, m, best_d)
        best_i = jnp.where(take, i, best_i)
    idx_ref[...] = best_i


def _dec_body(z_ref, q_ref, wd1_ref, bd1_ref, wd2_ref, bd2_ref,
              quant_ref, out_ref):
    z = z_ref[...]
    q = z + (q_ref[...] - z)
    quant_ref[...] = q
    h = jnp.maximum(
        jnp.dot(q, wd1_ref[...], preferred_element_type=jnp.float32)
        + bd1_ref[...], 0.0)
    out_ref[...] = (jnp.dot(h, wd2_ref[...], preferred_element_type=jnp.float32)
                    + bd2_ref[...])


_enc_call = pl.pallas_call(
    _enc_body,
    grid=(NT,),
    in_specs=[
        pl.BlockSpec((TT, D), lambda i: (i, 0)),
        pl.BlockSpec((D, HID), lambda i: (0, 0)),
        pl.BlockSpec((1, HID), lambda i: (0, 0)),
        pl.BlockSpec((HID, CODE_DIM), lambda i: (0, 0)),
        pl.BlockSpec((1, CODE_DIM), lambda i: (0, 0)),
        pl.BlockSpec((CODE_DIM, K), lambda i: (0, 0)),
        pl.BlockSpec((1, K), lambda i: (0, 0)),
    ],
    out_specs=[
        pl.BlockSpec((TT, CODE_DIM), lambda i: (i, 0)),
        pl.BlockSpec((TT, 1), lambda i: (i, 0)),
    ],
    out_shape=[
        jax.ShapeDtypeStruct((N, CODE_DIM), jnp.float32),
        jax.ShapeDtypeStruct((N, 1), jnp.int32),
    ],
    compiler_params=pltpu.CompilerParams(
        dimension_semantics=("arbitrary",)),
)

_dec_call = pl.pallas_call(
    _dec_body,
    grid=(NT,),
    in_specs=[
        pl.BlockSpec((TT, CODE_DIM), lambda i: (i, 0)),
        pl.BlockSpec((TT, CODE_DIM), lambda i: (i, 0)),
        pl.BlockSpec((CODE_DIM, HID), lambda i: (0, 0)),
        pl.BlockSpec((1, HID), lambda i: (0, 0)),
        pl.BlockSpec((HID, D), lambda i: (0, 0)),
        pl.BlockSpec((1, D), lambda i: (0, 0)),
    ],
    out_specs=[
        pl.BlockSpec((TT, CODE_DIM), lambda i: (i, 0)),
        pl.BlockSpec((TT, D), lambda i: (i, 0)),
    ],
    out_shape=[
        jax.ShapeDtypeStruct((N, CODE_DIM), jnp.float32),
        jax.ShapeDtypeStruct((N, D), jnp.float32),
    ],
    compiler_params=pltpu.CompilerParams(
        dimension_semantics=("arbitrary",)),
)


GW = 128


def _sc_gather_body(table_hbm, idx_hbm, out_hbm, idx_v, rows_v, sem):
    wid = lax.axis_index("s") * SC_CORES + lax.axis_index("c")
    base = wid * BPW
    pltpu.sync_copy(idx_hbm.at[pl.ds(base, BPW)], idx_v)
    pltpu.async_copy(table_hbm.at[idx_v], rows_v, sem).wait()
    pltpu.sync_copy(rows_v, out_hbm.at[pl.ds(base, BPW)])


@functools.cache
def _sc_gather_call():
    return pl.kernel(
        _sc_gather_body,
        mesh=plsc.VectorSubcoreMesh(core_axis_name="c", subcore_axis_name="s"),
        out_type=jax.ShapeDtypeStruct((N, GW), jnp.float32),
        scratch_types=[
            pltpu.VMEM((BPW,), jnp.int32),
            pltpu.VMEM((BPW, GW), jnp.float32),
            pltpu.SemaphoreType.DMA,
        ],
    )


def kernel(x, W_enc1, b_enc1, W_enc2, b_enc2, codeblocks,
           W_dec1, b_dec1, W_dec2, b_dec2):
    Bx = x.shape[0]
    patches = (x.reshape(Bx, CIN, HP, P, HP, P)
               .transpose(0, 2, 4, 1, 3, 5)
               .reshape(Bx * HP * HP, D))
    cbt = codeblocks.T
    cnorm = jnp.sum(codeblocks ** 2, axis=1).reshape(1, K)
    z, idx = _enc_call(patches, W_enc1, b_enc1.reshape(1, HID),
                       W_enc2, b_enc2.reshape(1, CODE_DIM), cbt, cnorm)
    nearest = idx.reshape(N)
    table_pad = jnp.pad(codeblocks, ((0, 0), (0, GW - CODE_DIM)))
    q_raw = _sc_gather_call()(table_pad, nearest)[:, :CODE_DIM]
    quant, d2 = _dec_call(z, q_raw, W_dec1, b_dec1.reshape(1, HID),
                          W_dec2, b_dec2.reshape(1, D))
    dec = (d2.reshape(Bx, HP, HP, CIN, P, P)
           .transpose(0, 3, 1, 4, 2, 5)
           .reshape(Bx, CIN, HW, HW))
    return (dec, z, quant)
